# baseline (speedup 1.0000x reference)
import jax
import jax.numpy as jnp
from jax.experimental import pallas as pl
from jax.experimental.pallas import tpu as pltpu

_DIM = 768
_H = 12
_K = 90
_B = 8
_N = 576
_HD = _DIM // _H
_SCALE = _HD ** -0.5
_MININT = -(2 ** 31)
_MAXINT = 2 ** 31 - 1


def _qkv_kernel(x_ref, w_ref, o_ref):
    o_ref[0, 0] = jax.lax.dot_general(
        x_ref[0], w_ref[...],
        dimension_numbers=(((1,), (1,)), ((), ())),
        preferred_element_type=jnp.float32)


def _attn_kernel(islast_ref, q_ref, k_ref, v_ref, attn_ref, ho_ref):
    q = q_ref[0, 0]
    k = k_ref[0, 0]
    v = v_ref[0, 0]
    s = jax.lax.dot_general(
        q, k, dimension_numbers=(((1,), (1,)), ((), ())),
        preferred_element_type=jnp.float32) * _SCALE

    bits = jax.lax.bitcast_convert_type(s, jnp.int32)
    su = bits ^ ((bits >> 31) & 0x7FFFFFFF)

    maxf = jnp.max(s, axis=1, keepdims=True)
    minf = jnp.min(s, axis=1, keepdims=True)

    def _f2key(f):
        fb = jax.lax.bitcast_convert_type(f, jnp.int32)
        return fb ^ ((fb >> 31) & 0x7FFFFFFF)

    L0 = _f2key(minf)
    H0 = _f2key(maxf)
    krem0 = jnp.full((_N, 1), _K, jnp.int32)
    m0 = jnp.full((_N, 1), _N, jnp.int32)
    done0 = (L0 >= H0).astype(jnp.int32)
    casemin0 = done0
    alldone0 = jnp.min(done0)

    def cond(carry):
        i, L, H, krem, m, done, casemin, alldone = carry
        return jnp.logical_and(i < 72, alldone == 0)

    def body(carry):
        i, L, H, krem, m, done, casemin, alldone = carry
        undone = done == 0
        Lf = L.astype(jnp.float32)
        Hf = H.astype(jnp.float32)
        frac = 1.0 - krem.astype(jnp.float32) / m.astype(jnp.float32)
        tfrac = jnp.where(i % 2 == 0, frac, 0.5)
        M = (Lf + (Hf - Lf) * tfrac).astype(jnp.int32)
        M = jnp.clip(M, L + 1, H)
        inc = jnp.logical_and(su >= M, su <= H)
        c = jnp.sum(inc.astype(jnp.int32), axis=1, keepdims=True)
        ge = c >= krem
        Ln = jnp.where(ge, M, L)
        Hn = jnp.where(ge, H, M - 1)
        kn = jnp.where(ge, krem, krem - c)
        mn = jnp.where(ge, c, m - c)
        L = jnp.where(undone, Ln, L)
        H = jnp.where(undone, Hn, H)
        krem = jnp.where(undone, kn, krem)
        m = jnp.where(undone, mn, m)
        fin_min = jnp.logical_or(krem == m, L >= H).astype(jnp.int32)
        newdone = jnp.maximum(fin_min, (krem == 1).astype(jnp.int32))
        casemin = jnp.where(undone, fin_min, casemin)
        done = jnp.maximum(done, newdone)
        alldone = jnp.min(done)
        return (i + 1, L, H, krem, m, done, casemin, alldone)

    (_, L, H, krem, m_cnt, done, casemin, _) = jax.lax.while_loop(
        cond, body, (jnp.int32(0), L0, H0, krem0, m0, done0, casemin0,
                     alldone0))

    inclass = jnp.logical_and(su >= L, su <= H)
    mn_k = jnp.min(jnp.where(inclass, su, _MAXINT), axis=1, keepdims=True)
    mx_k = jnp.max(jnp.where(inclass, su, _MININT), axis=1, keepdims=True)
    sthr = jnp.where(casemin != 0, mn_k, mx_k)
    sthr = jnp.where(islast_ref[0] == 0, sthr, _MININT)
    mask = su >= sthr

    m = maxf
    p = jnp.where(mask, jnp.exp(s - m), 0.0)
    a = p / jnp.sum(p, axis=1, keepdims=True)
    attn_ref[0, 0] = a
    ho_ref[0, 0] = jax.lax.dot_general(
        a, v, dimension_numbers=(((1,), (0,)), ((), ())),
        preferred_element_type=jnp.float32)


def _proj_kernel(ho_ref, w_ref, b_ref, o_ref):
    o_ref[0] = jax.lax.dot_general(
        ho_ref[0], w_ref[...],
        dimension_numbers=(((1,), (1,)), ((), ())),
        preferred_element_type=jnp.float32) + b_ref[...]


def kernel(x, W_qkv, W_proj, b_proj, islast):
    islast_arr = jnp.asarray(islast, jnp.int32).reshape(1)

    qkv = pl.pallas_call(
        _qkv_kernel,
        grid=(_B, 3 * _H),
        in_specs=[
            pl.BlockSpec((1, _N, _DIM), lambda b, g: (b, 0, 0)),
            pl.BlockSpec((_HD, _DIM), lambda b, g: (g, 0)),
        ],
        out_specs=pl.BlockSpec((1, 1, _N, _HD), lambda b, g: (b, g, 0, 0)),
        out_shape=jax.ShapeDtypeStruct((_B, 3 * _H, _N, _HD), jnp.float32),
        compiler_params=pltpu.CompilerParams(
            dimension_semantics=("parallel", "parallel")),
    )(x, W_qkv)

    attn, ho = pl.pallas_call(
        _attn_kernel,
        grid=(_B, _H),
        in_specs=[
            pl.BlockSpec(memory_space=pltpu.SMEM),
            pl.BlockSpec((1, 1, _N, _HD), lambda b, h: (b, h, 0, 0)),
            pl.BlockSpec((1, 1, _N, _HD), lambda b, h: (b, _H + h, 0, 0)),
            pl.BlockSpec((1, 1, _N, _HD), lambda b, h: (b, 2 * _H + h, 0, 0)),
        ],
        out_specs=[
            pl.BlockSpec((1, 1, _N, _N), lambda b, h: (b, h, 0, 0)),
            pl.BlockSpec((1, 1, _N, _HD), lambda b, h: (b, h, 0, 0)),
        ],
        out_shape=[
            jax.ShapeDtypeStruct((_B, _H, _N, _N), jnp.float32),
            jax.ShapeDtypeStruct((_B, _H, _N, _HD), jnp.float32),
        ],
        compiler_params=pltpu.CompilerParams(
            dimension_semantics=("parallel", "parallel")),
    )(islast_arr, qkv, qkv, qkv)

    ho_bnc = ho.transpose(0, 2, 1, 3).reshape(_B, _N, _DIM)

    out = pl.pallas_call(
        _proj_kernel,
        grid=(_B,),
        in_specs=[
            pl.BlockSpec((1, _N, _DIM), lambda b: (b, 0, 0)),
            pl.BlockSpec((_DIM, _DIM), lambda b: (0, 0)),
            pl.BlockSpec((1, _DIM), lambda b: (0, 0)),
        ],
        out_specs=pl.BlockSpec((1, _N, _DIM), lambda b: (b, 0, 0)),
        out_shape=jax.ShapeDtypeStruct((_B, _N, _DIM), jnp.float32),
        compiler_params=pltpu.CompilerParams(
            dimension_semantics=("parallel",)),
    )(ho_bnc, W_proj, b_proj.reshape(1, _DIM))

    return (out, attn)


# speedup vs baseline: 1.3340x; 1.3340x over previous
import jax
import jax.numpy as jnp
from jax.experimental import pallas as pl
from jax.experimental.pallas import tpu as pltpu

_DIM = 768
_H = 12
_K = 90
_B = 8
_N = 576
_HD = _DIM // _H
_SCALE = _HD ** -0.5
_MININT = -(2 ** 31)
_MAXINT = 2 ** 31 - 1


def _qkv_kernel(x_ref, w_ref, o_ref):
    o_ref[0, 0] = jax.lax.dot_general(
        x_ref[0], w_ref[...],
        dimension_numbers=(((1,), (1,)), ((), ())),
        preferred_element_type=jnp.float32)


def _attn_kernel(islast_ref, q_ref, k_ref, v_ref, attn_ref, ho_ref):
    q = q_ref[0, 0]
    k = k_ref[0, 0]
    v = v_ref[0, 0]
    s = jax.lax.dot_general(
        q, k, dimension_numbers=(((1,), (1,)), ((), ())),
        preferred_element_type=jnp.float32) * _SCALE

    def _key2f(t):
        st = t ^ _MININT
        fb = st ^ ((st >> 31) & 0x7FFFFFFF)
        return jax.lax.bitcast_convert_type(fb, jnp.float32)

    def body(i, T):
        b = 31 - i
        cand = T | (jnp.int32(1) << b)
        thrf = _key2f(cand)
        c = jnp.sum((s >= thrf).astype(jnp.int32), axis=1, keepdims=True)
        return jnp.where(c >= _K, cand, T)

    T = jax.lax.fori_loop(0, 32, body, jnp.zeros((_N, 1), jnp.int32))

    thr = _key2f(T)
    thr = jnp.where(islast_ref[0] == 0, thr, -jnp.inf)
    mask = s >= thr

    m = jnp.max(s, axis=1, keepdims=True)
    p = jnp.where(mask, jnp.exp(s - m), 0.0)
    a = p / jnp.sum(p, axis=1, keepdims=True)
    attn_ref[0, 0] = a
    ho_ref[0, 0] = jax.lax.dot_general(
        a, v, dimension_numbers=(((1,), (0,)), ((), ())),
        preferred_element_type=jnp.float32)


def _proj_kernel(ho_ref, w_ref, b_ref, o_ref):
    o_ref[0] = jax.lax.dot_general(
        ho_ref[0], w_ref[...],
        dimension_numbers=(((1,), (1,)), ((), ())),
        preferred_element_type=jnp.float32) + b_ref[...]


def kernel(x, W_qkv, W_proj, b_proj, islast):
    islast_arr = jnp.asarray(islast, jnp.int32).reshape(1)

    qkv = pl.pallas_call(
        _qkv_kernel,
        grid=(_B, 3 * _H),
        in_specs=[
            pl.BlockSpec((1, _N, _DIM), lambda b, g: (b, 0, 0)),
            pl.BlockSpec((_HD, _DIM), lambda b, g: (g, 0)),
        ],
        out_specs=pl.BlockSpec((1, 1, _N, _HD), lambda b, g: (b, g, 0, 0)),
        out_shape=jax.ShapeDtypeStruct((_B, 3 * _H, _N, _HD), jnp.float32),
        compiler_params=pltpu.CompilerParams(
            dimension_semantics=("parallel", "parallel")),
    )(x, W_qkv)

    attn, ho = pl.pallas_call(
        _attn_kernel,
        grid=(_B, _H),
        in_specs=[
            pl.BlockSpec(memory_space=pltpu.SMEM),
            pl.BlockSpec((1, 1, _N, _HD), lambda b, h: (b, h, 0, 0)),
            pl.BlockSpec((1, 1, _N, _HD), lambda b, h: (b, _H + h, 0, 0)),
            pl.BlockSpec((1, 1, _N, _HD), lambda b, h: (b, 2 * _H + h, 0, 0)),
        ],
        out_specs=[
            pl.BlockSpec((1, 1, _N, _N), lambda b, h: (b, h, 0, 0)),
            pl.BlockSpec((1, 1, _N, _HD), lambda b, h: (b, h, 0, 0)),
        ],
        out_shape=[
            jax.ShapeDtypeStruct((_B, _H, _N, _N), jnp.float32),
            jax.ShapeDtypeStruct((_B, _H, _N, _HD), jnp.float32),
        ],
        compiler_params=pltpu.CompilerParams(
            dimension_semantics=("parallel", "parallel")),
    )(islast_arr, qkv, qkv, qkv)

    ho_bnc = ho.transpose(0, 2, 1, 3).reshape(_B, _N, _DIM)

    out = pl.pallas_call(
        _proj_kernel,
        grid=(_B,),
        in_specs=[
            pl.BlockSpec((1, _N, _DIM), lambda b: (b, 0, 0)),
            pl.BlockSpec((_DIM, _DIM), lambda b: (0, 0)),
            pl.BlockSpec((1, _DIM), lambda b: (0, 0)),
        ],
        out_specs=pl.BlockSpec((1, _N, _DIM), lambda b: (b, 0, 0)),
        out_shape=jax.ShapeDtypeStruct((_B, _N, _DIM), jnp.float32),
        compiler_params=pltpu.CompilerParams(
            dimension_semantics=("parallel",)),
    )(ho_bnc, W_proj, b_proj.reshape(1, _DIM))

    return (out, attn)


# speedup vs baseline: 1.4981x; 1.1230x over previous
import jax
import jax.numpy as jnp
from jax.experimental import pallas as pl
from jax.experimental.pallas import tpu as pltpu

_DIM = 768
_H = 12
_K = 90
_B = 8
_N = 576
_HD = _DIM // _H
_SCALE = _HD ** -0.5
_MININT = -(2 ** 31)


def _key2f(t):
    st = t ^ _MININT
    fb = st ^ ((st >> 31) & 0x7FFFFFFF)
    return jax.lax.bitcast_convert_type(fb, jnp.float32)


def _fused_kernel(islast_ref, x_ref, wqkv_ref, wpt_ref, bias_ref,
                  attn_ref, out_ref):
    h = pl.program_id(1)
    x = x_ref[0]
    wq = wqkv_ref[pl.ds(h * _HD, _HD), :]
    wk = wqkv_ref[pl.ds((_H + h) * _HD, _HD), :]
    wv = wqkv_ref[pl.ds((2 * _H + h) * _HD, _HD), :]
    cdims = (((1,), (1,)), ((), ()))
    q = jax.lax.dot_general(x, wq, cdims, preferred_element_type=jnp.float32)
    k = jax.lax.dot_general(x, wk, cdims, preferred_element_type=jnp.float32)
    v = jax.lax.dot_general(x, wv, cdims, preferred_element_type=jnp.float32)
    s = jax.lax.dot_general(
        q, k, cdims, preferred_element_type=jnp.float32) * _SCALE

    def body(i, T):
        b = 31 - i
        cand = T | (jnp.int32(1) << b)
        thrf = _key2f(cand)
        c = jnp.sum((s >= thrf).astype(jnp.int32), axis=1, keepdims=True)
        return jnp.where(c >= _K, cand, T)

    T = jax.lax.fori_loop(0, 32, body, jnp.zeros((_N, 1), jnp.int32))

    thr = _key2f(T)
    thr = jnp.where(islast_ref[0] == 0, thr, -jnp.inf)
    mask = s >= thr

    m = jnp.max(s, axis=1, keepdims=True)
    p = jnp.where(mask, jnp.exp(s - m), 0.0)
    a = p / jnp.sum(p, axis=1, keepdims=True)
    attn_ref[0, 0] = a

    av = jax.lax.dot_general(
        a, v, dimension_numbers=(((1,), (0,)), ((), ())),
        preferred_element_type=jnp.float32)
    wpt = wpt_ref[pl.ds(h * _HD, _HD), :]
    contrib = jax.lax.dot_general(
        av, wpt, dimension_numbers=(((1,), (0,)), ((), ())),
        preferred_element_type=jnp.float32)

    @pl.when(h == 0)
    def _init():
        out_ref[0] = contrib + bias_ref[...]

    @pl.when(h != 0)
    def _acc():
        out_ref[0] += contrib


def kernel(x, W_qkv, W_proj, b_proj, islast):
    islast_arr = jnp.asarray(islast, jnp.int32).reshape(1)

    attn, out = pl.pallas_call(
        _fused_kernel,
        grid=(_B, _H),
        in_specs=[
            pl.BlockSpec(memory_space=pltpu.SMEM),
            pl.BlockSpec((1, _N, _DIM), lambda b, h: (b, 0, 0)),
            pl.BlockSpec((3 * _DIM, _DIM), lambda b, h: (0, 0)),
            pl.BlockSpec((_DIM, _DIM), lambda b, h: (0, 0)),
            pl.BlockSpec((1, _DIM), lambda b, h: (0, 0)),
        ],
        out_specs=[
            pl.BlockSpec((1, 1, _N, _N), lambda b, h: (b, h, 0, 0)),
            pl.BlockSpec((1, _N, _DIM), lambda b, h: (b, 0, 0)),
        ],
        out_shape=[
            jax.ShapeDtypeStruct((_B, _H, _N, _N), jnp.float32),
            jax.ShapeDtypeStruct((_B, _N, _DIM), jnp.float32),
        ],
        compiler_params=pltpu.CompilerParams(
            dimension_semantics=("parallel", "arbitrary")),
    )(islast_arr, x, W_qkv, W_proj.T, b_proj.reshape(1, _DIM))

    return (out, attn)


# speedup vs baseline: 2.4064x; 1.6063x over previous
import jax
import jax.numpy as jnp
from jax.experimental import pallas as pl
from jax.experimental.pallas import tpu as pltpu

_DIM = 768
_H = 12
_K = 90
_B = 8
_N = 576
_HD = _DIM // _H
_SCALE = _HD ** -0.5
_MININT = -(2 ** 31)


def _key2f(t):
    st = t ^ _MININT
    fb = st ^ ((st >> 31) & 0x7FFFFFFF)
    return jax.lax.bitcast_convert_type(fb, jnp.float32)


def _fused_kernel(islast_ref, x_ref, wqkv_ref, wpt_ref, bias_ref,
                  attn_ref, out_ref):
    h = pl.program_id(1)
    x = x_ref[0]
    wq = wqkv_ref[pl.ds(h * _HD, _HD), :]
    wk = wqkv_ref[pl.ds((_H + h) * _HD, _HD), :]
    wv = wqkv_ref[pl.ds((2 * _H + h) * _HD, _HD), :]
    cdims = (((1,), (1,)), ((), ()))
    q = jax.lax.dot_general(x, wq, cdims, preferred_element_type=jnp.float32)
    k = jax.lax.dot_general(x, wk, cdims, preferred_element_type=jnp.float32)
    v = jax.lax.dot_general(x, wv, cdims, preferred_element_type=jnp.float32)
    st = jax.lax.dot_general(
        k, q, cdims, preferred_element_type=jnp.float32) * _SCALE

    def body(i, T):
        b = 31 - i
        cand = T | (jnp.int32(1) << b)
        thrf = _key2f(cand)
        c = jnp.sum((st >= thrf).astype(jnp.int32), axis=0, keepdims=True)
        return jnp.where(c >= _K, cand, T)

    T = jax.lax.fori_loop(0, 32, body, jnp.zeros((1, _N), jnp.int32))

    thr = _key2f(T)
    thr = jnp.where(islast_ref[0] == 0, thr, -jnp.inf)
    mask = st >= thr

    m = jnp.max(st, axis=0, keepdims=True)
    p = jnp.where(mask, jnp.exp(st - m), 0.0)
    a = p / jnp.sum(p, axis=0, keepdims=True)
    attn_ref[0, 0] = a.T

    av = jax.lax.dot_general(
        a, v, dimension_numbers=(((0,), (0,)), ((), ())),
        preferred_element_type=jnp.float32)
    wpt = wpt_ref[pl.ds(h * _HD, _HD), :]
    contrib = jax.lax.dot_general(
        av, wpt, dimension_numbers=(((1,), (0,)), ((), ())),
        preferred_element_type=jnp.float32)

    @pl.when(h == 0)
    def _init():
        out_ref[0] = contrib + bias_ref[...]

    @pl.when(h != 0)
    def _acc():
        out_ref[0] += contrib


def kernel(x, W_qkv, W_proj, b_proj, islast):
    islast_arr = jnp.asarray(islast, jnp.int32).reshape(1)

    attn, out = pl.pallas_call(
        _fused_kernel,
        grid=(_B, _H),
        in_specs=[
            pl.BlockSpec(memory_space=pltpu.SMEM),
            pl.BlockSpec((1, _N, _DIM), lambda b, h: (b, 0, 0)),
            pl.BlockSpec((3 * _DIM, _DIM), lambda b, h: (0, 0)),
            pl.BlockSpec((_DIM, _DIM), lambda b, h: (0, 0)),
            pl.BlockSpec((1, _DIM), lambda b, h: (0, 0)),
        ],
        out_specs=[
            pl.BlockSpec((1, 1, _N, _N), lambda b, h: (b, h, 0, 0)),
            pl.BlockSpec((1, _N, _DIM), lambda b, h: (b, 0, 0)),
        ],
        out_shape=[
            jax.ShapeDtypeStruct((_B, _H, _N, _N), jnp.float32),
            jax.ShapeDtypeStruct((_B, _N, _DIM), jnp.float32),
        ],
        compiler_params=pltpu.CompilerParams(
            dimension_semantics=("parallel", "arbitrary")),
    )(islast_arr, x, W_qkv, W_proj.T, b_proj.reshape(1, _DIM))

    return (out, attn)
